# baseline (speedup 1.0000x reference)
import functools

import jax
import jax.numpy as jnp
from jax import lax
from jax.experimental import pallas as pl
from jax.experimental.pallas import tpu as pltpu
from jax.experimental.pallas import tpu_sc as plsc

N = 10000
E = 320000
D = 128
NPAD = 10240
NC = 2
NS = 16
NW = NC * NS
EW = E // NW
B = 80
EW2 = 10080
NCHUNK = EW2 // B
CH = NPAD // NW
NACC = NPAD
RPW = NACC // NS
ZR = 64

_f32 = jnp.float32
_i32 = jnp.int32

_MESH = dict(core_axis_name="c", subcore_axis_name="s", num_cores=NC,
             num_subcores=NS)

_GDN = lax.GatherDimensionNumbers(offset_dims=(), collapsed_slice_dims=(0,),
                                  start_index_map=(0,))


def _lane_bcast(vec, i):
    idx = jnp.full((16, 1), i, _i32)
    return lax.gather(vec, idx, _GDN, (1,),
                      mode=lax.GatherScatterMode.PROMISE_IN_BOUNDS)


def _wid():
    return lax.axis_index("s") * NC + lax.axis_index("c")


def _tc_zst_body(h_ref, wfc_ref, wa_ref, z_ref, st_ref):
    hb = h_ref[...]
    z = lax.dot_general(hb, wfc_ref[...], (((1,), (1,)), ((), ())),
                        preferred_element_type=_f32)
    z = jnp.where(z >= 0, z, 0.01 * z)
    z_ref[...] = z
    st = lax.dot_general(wa_ref[...], z, (((1,), (1,)), ((), ())),
                         preferred_element_type=_f32)
    st_ref[...] = jnp.concatenate(
        [st, jnp.zeros((6, st.shape[1]), _f32)], axis=0)


def _tc_zst(h_pad, wfc, wa2):
    BN = 512
    grid = NPAD // BN
    return pl.pallas_call(
        _tc_zst_body,
        grid=(grid,),
        in_specs=[
            pl.BlockSpec((BN, D), lambda i: (i, 0)),
            pl.BlockSpec((D, D), lambda i: (0, 0)),
            pl.BlockSpec((2, D), lambda i: (0, 0)),
        ],
        out_specs=[
            pl.BlockSpec((BN, D), lambda i: (i, 0)),
            pl.BlockSpec((8, BN), lambda i: (0, i)),
        ],
        out_shape=[
            jax.ShapeDtypeStruct((NPAD, D), _f32),
            jax.ShapeDtypeStruct((8, NPAD), _f32),
        ],
    )(h_pad, wfc, wa2)


def _tc_sum_body(p_ref, o_ref):
    o_ref[...] = p_ref[0] + p_ref[1]


def _tc_sum(part):
    BN = 512
    return pl.pallas_call(
        _tc_sum_body,
        grid=(NACC // BN,),
        in_specs=[pl.BlockSpec((2, BN, D), lambda i: (0, i, 0))],
        out_specs=pl.BlockSpec((BN, D), lambda i: (i, 0)),
        out_shape=jax.ShapeDtypeStruct((NACC, D), _f32),
    )(part)


@functools.partial(
    pl.kernel,
    out_type=(jax.ShapeDtypeStruct((E,), _f32),
              jax.ShapeDtypeStruct((NW * NPAD,), _f32),
              jax.ShapeDtypeStruct((NW * NPAD,), _f32)),
    mesh=plsc.VectorSubcoreMesh(**_MESH),
    compiler_params=pltpu.CompilerParams(needs_layout_passes=False),
    scratch_types=[
        pltpu.VMEM((NPAD,), _f32),
        pltpu.VMEM((NPAD,), _f32),
        pltpu.VMEM((NPAD,), _f32),
        pltpu.VMEM((NPAD,), _f32),
        pltpu.VMEM((EW,), _i32),
        pltpu.VMEM((EW,), _i32),
        pltpu.VMEM((EW,), _f32),
        pltpu.VMEM((EW,), _f32),
        pltpu.VMEM((16,), _f32),
        pltpu.VMEM((16,), _f32),
        pltpu.SemaphoreType.DMA,
    ],
)
def _k1(src_h, dst_h, ea_h, s_h, t_h, we_h, wm_h, exl_h, pmax_h, pden_h,
        s_v, t_v, m_v, d_v, src_v, dst_v, ea_v, ne_v, we_v, wm_v, sem):
    wid = _wid()
    base = wid * EW
    descs = [
        pltpu.async_copy(s_h, s_v, sem),
        pltpu.async_copy(t_h, t_v, sem),
        pltpu.async_copy(src_h.at[pl.ds(base, EW)], src_v, sem),
        pltpu.async_copy(dst_h.at[pl.ds(base, EW)], dst_v, sem),
        pltpu.async_copy(ea_h.at[pl.ds(base, EW)], ea_v, sem),
        pltpu.async_copy(we_h, we_v, sem),
        pltpu.async_copy(wm_h, wm_v, sem),
    ]

    def init_body(i, _):
        m_v[pl.ds(i * 16, 16)] = jnp.full((16,), -1e30, _f32)
        d_v[pl.ds(i * 16, 16)] = jnp.zeros((16,), _f32)
        return 0
    lax.fori_loop(0, NPAD // 16, init_body, 0)
    for de in descs:
        de.wait()
    we = we_v[...]
    wm = wm_v[...]

    def edge_body(i, _):
        sl = pl.ds(i * 16, 16)
        src16 = src_v[sl]
        dst16 = dst_v[sl]
        a = plsc.load_gather(s_v, [src16]) + plsc.load_gather(t_v, [dst16])
        x = a * ea_v[sl] * we
        ne = jnp.maximum(x, 0.01 * x) * wm
        ne_v[sl] = ne
        cur = plsc.load_gather(m_v, [dst16])
        need = ne > cur
        plsc.store_scatter(m_v, [dst16], ne, mask=need)

        def cond(p):
            return jnp.any(p)

        def body(p):
            cur2 = plsc.load_gather(m_v, [dst16])
            need2 = jnp.logical_and(p, ne > cur2)
            plsc.store_scatter(m_v, [dst16], ne, mask=need2)
            return need2
        lax.while_loop(cond, body, need)
        return 0
    lax.fori_loop(0, EW // 16, edge_body, 0)

    def exp_body(i, _):
        sl = pl.ds(i * 16, 16)
        dst16 = dst_v[sl]
        ex = jnp.exp(ne_v[sl] - plsc.load_gather(m_v, [dst16]))
        ne_v[sl] = ex
        plsc.addupdate_scatter(d_v, [dst16], ex)
        return 0
    lax.fori_loop(0, EW // 16, exp_body, 0)

    pltpu.sync_copy(ne_v, exl_h.at[pl.ds(base, EW)])
    pltpu.sync_copy(m_v, pmax_h.at[pl.ds(wid * NPAD, NPAD)])
    pltpu.sync_copy(d_v, pden_h.at[pl.ds(wid * NPAD, NPAD)])


CH2 = 128
NH = NPAD // NS // CH2


@functools.partial(
    pl.kernel,
    out_type=jax.ShapeDtypeStruct((NW * EW2,), _f32),
    mesh=plsc.VectorSubcoreMesh(**_MESH),
    compiler_params=pltpu.CompilerParams(needs_layout_passes=False),
    scratch_types=[
        pltpu.VMEM((NW * CH2,), _f32),
        pltpu.VMEM((NW * CH2,), _f32),
        pltpu.VMEM((NW * CH2,), _f32),
        pltpu.VMEM((NPAD,), _f32),
        pltpu.VMEM((EW,), _i32),
        pltpu.VMEM((EW,), _f32),
        pltpu.VMEM((EW2,), _f32),
        pltpu.VMEM_SHARED((NW * NPAD,), _f32),
        pltpu.SemaphoreType.DMA,
        pltpu.SemaphoreType.DMA,
    ],
)
def _k4(exl_h, dst_h, pm_h, pd_h, al_h,
        bm_v, bd_v, sc_v, scl_v, dst_v, ex_v, al_v, scale_sh, sem, sem2):
    cid = lax.axis_index("c")
    sid = lax.axis_index("s")
    wid = sid * NC + cid
    base = wid * EW
    edescs = [
        pltpu.async_copy(dst_h.at[pl.ds(base, EW)], dst_v, sem2),
        pltpu.async_copy(exl_h.at[pl.ds(base, EW)], ex_v, sem2),
    ]
    for h in range(NH):
        n0 = sid * (NH * CH2) + h * CH2
        descs = []
        for r in range(NW):
            descs.append(pltpu.async_copy(
                pm_h.at[pl.ds(r * NPAD + n0, CH2)],
                bm_v.at[pl.ds(r * CH2, CH2)], sem))
            descs.append(pltpu.async_copy(
                pd_h.at[pl.ds(r * NPAD + n0, CH2)],
                bd_v.at[pl.ds(r * CH2, CH2)], sem))
        for de in descs:
            de.wait()

        def body(j, _):
            sl = pl.ds(j * 16, 16)
            mx = bm_v[sl]
            for r in range(1, NW):
                mx = jnp.maximum(mx, bm_v[pl.ds(r * CH2 + j * 16, 16)])
            den = jnp.zeros((16,), _f32)
            for r in range(NW):
                rsl = pl.ds(r * CH2 + j * 16, 16)
                den = den + jnp.exp(bm_v[rsl] - mx) * bd_v[rsl]
            den = jnp.where(den > 0.0, den, 1.0)
            for r in range(NW):
                rsl = pl.ds(r * CH2 + j * 16, 16)
                sc_v[rsl] = jnp.exp(bm_v[rsl] - mx) / den
            return 0
        lax.fori_loop(0, CH2 // 16, body, 0)
        for r in range(NW):
            pltpu.sync_copy(sc_v.at[pl.ds(r * CH2, CH2)],
                            scale_sh.at[pl.ds(r * NPAD + n0, CH2)])
    plsc.subcore_barrier()

    pltpu.sync_copy(scale_sh.at[pl.ds(wid * NPAD, NPAD)], scl_v)
    for i in range((EW2 - EW) // 16):
        al_v[pl.ds(EW + i * 16, 16)] = jnp.zeros((16,), _f32)
    for de in edescs:
        de.wait()

    def body2(i, _):
        sl = pl.ds(i * 16, 16)
        dst16 = dst_v[sl]
        al_v[sl] = ex_v[sl] * plsc.load_gather(scl_v, [dst16])
        return 0
    lax.fori_loop(0, EW // 16, body2, 0)
    pltpu.sync_copy(al_v, al_h.at[pl.ds(wid * EW2, EW2)])


@functools.partial(
    pl.kernel,
    out_type=jax.ShapeDtypeStruct((NC, NACC, D), _f32),
    mesh=plsc.VectorSubcoreMesh(**_MESH),
    compiler_params=pltpu.CompilerParams(needs_layout_passes=False),
    scratch_types=[
        pltpu.VMEM((EW2,), _f32),
        pltpu.VMEM((2, B), _i32),
        pltpu.VMEM((2, B), _i32),
        pltpu.VMEM((B, D), _f32),
        pltpu.VMEM((B, D), _f32),
        pltpu.VMEM_SHARED((NACC, D), _f32),
        pltpu.SemaphoreType.DMA,
        pltpu.SemaphoreType.DMA,
        pltpu.SemaphoreType.DMA,
        pltpu.SemaphoreType.DMA,
        pltpu.SemaphoreType.DMA,
        pltpu.SemaphoreType.DMA,
        pltpu.SemaphoreType.DMA,
        pltpu.SemaphoreType.DMA,
    ],
)
def _k5(al_h, src_h, dst_h, z_h, part_h,
        al_v, sidx_v, didx_v, rows0_v, rows1_v, acc_sh,
        gs0, gs1, ss0, ss1, is0, is1, ds0, ds1):
    cid = lax.axis_index("c")
    sid = lax.axis_index("s")
    wid = sid * NC + cid
    rows = (rows0_v, rows1_v)
    gsem = (gs0, gs1)
    ssem = (ss0, ss1)
    isem = (is0, is1)
    dsem = (ds0, ds1)
    base2 = wid * EW2

    def _erow(h, j):
        return h.at[pl.ds(base2 + j * B, B)]

    pltpu.sync_copy(al_h.at[pl.ds(base2, EW2)], al_v)
    for b in range(2):
        pltpu.sync_copy(_erow(src_h, b), sidx_v.at[b])
        pltpu.async_copy(_erow(dst_h, b), didx_v.at[b], dsem[b])

    def zinit(i, _):
        for r in range(ZR):
            rows0_v[r, pl.ds(i * 16, 16)] = jnp.zeros((16,), _f32)
        return 0
    lax.fori_loop(0, D // 16, zinit, 0)
    row0 = sid * RPW

    def zcopy(q, _):
        pltpu.sync_copy(rows0_v.at[pl.ds(0, ZR)],
                        acc_sh.at[pl.ds(row0 + q * ZR, ZR)])
        return 0
    lax.fori_loop(0, RPW // ZR, zcopy, 0)
    plsc.subcore_barrier()

    for b in range(2):
        pltpu.async_copy(z_h.at[sidx_v.at[b]], rows[b], gsem[b])

    def _scale(b, jb):
        for k in range(B // 16):
            al = al_v[pl.ds(jb * B + k * 16, 16)]
            for i in range(16):
                bc = _lane_bcast(al, i)
                r = k * 16 + i
                for c in range(D // 16):
                    cs = pl.ds(c * 16, 16)
                    rows[b][r, cs] = rows[b][r, cs] * bc

    def _pair(j, with_next):
        sdesc = [None, None]
        idescs = [None, None]
        for b in range(2):
            jb = j + b
            pltpu.make_async_copy(z_h.at[sidx_v.at[b]], rows[b],
                                  gsem[b]).wait()
            if with_next:
                idescs[b] = pltpu.async_copy(_erow(src_h, jb + 2),
                                             sidx_v.at[b], isem[b])
            _scale(b, jb)
            pltpu.make_async_copy(_erow(dst_h, 0), didx_v.at[b],
                                  dsem[b]).wait()
            sdesc[b] = pltpu.async_copy(rows[b], acc_sh.at[didx_v.at[b]],
                                        ssem[b], add=True)
        for b in range(2):
            sdesc[b].wait()
            if with_next:
                pltpu.async_copy(_erow(dst_h, j + b + 2), didx_v.at[b],
                                 dsem[b])
                idescs[b].wait()
                pltpu.async_copy(z_h.at[sidx_v.at[b]], rows[b], gsem[b])

    def pair_body(i, _):
        _pair(2 * i, True)
        return 0
    lax.fori_loop(0, NCHUNK // 2 - 1, pair_body, 0)
    _pair(NCHUNK - 2, False)

    plsc.subcore_barrier()

    def dump(q, _):
        sl = pl.ds(row0 + q * ZR, ZR)
        pltpu.sync_copy(acc_sh.at[sl], part_h.at[cid].at[sl])
        return 0
    lax.fori_loop(0, RPW // ZR, dump, 0)


def kernel(h, edge_index, edge_attr, W_fc, W_attn, W_edge, W_m):
    h_pad = jnp.pad(h, ((0, NPAD - N), (0, 0)))
    z, st8 = _tc_zst(h_pad, W_fc, W_attn.reshape(2, D))
    s = st8[0]
    t = st8[1]
    src = edge_index[0]
    dst = edge_index[1]
    ea = edge_attr[:, 0]
    we = jnp.broadcast_to(W_edge.reshape(()), (16,)).astype(_f32)
    wm = jnp.broadcast_to(W_m.reshape(()), (16,)).astype(_f32)

    exl, pmax, pden = _k1(src, dst, ea, s, t, we, wm)
    al2 = _k4(exl, dst, pmax, pden)

    npad_e = EW2 - EW
    pad_idx = (jnp.arange(npad_e, dtype=_i32) * 97) % N
    pad_blk = jnp.broadcast_to(pad_idx, (NW, npad_e))

    def _pad_edges(x):
        return jnp.concatenate([x.reshape(NW, EW), pad_blk],
                               axis=1).reshape(-1)

    part = _k5(al2, _pad_edges(src), _pad_edges(dst), z)
    out = _tc_sum(part)
    return out[:N]


# speedup vs baseline: 1.0323x; 1.0323x over previous
import functools

import jax
import jax.numpy as jnp
from jax import lax
from jax.experimental import pallas as pl
from jax.experimental.pallas import tpu as pltpu
from jax.experimental.pallas import tpu_sc as plsc

N = 10000
E = 320000
D = 128
NPAD = 10240
NC = 2
NS = 16
NW = NC * NS
EW = E // NW
B = 80
EW2 = 10080
NCHUNK = EW2 // B
CH = NPAD // NW
NACC = NPAD
RPW = NACC // NS
ZR = 64

_f32 = jnp.float32
_i32 = jnp.int32

_MESH = dict(core_axis_name="c", subcore_axis_name="s", num_cores=NC,
             num_subcores=NS)

_GDN = lax.GatherDimensionNumbers(offset_dims=(), collapsed_slice_dims=(0,),
                                  start_index_map=(0,))


def _lane_bcast(vec, i):
    idx = jnp.full((16, 1), i, _i32)
    return lax.gather(vec, idx, _GDN, (1,),
                      mode=lax.GatherScatterMode.PROMISE_IN_BOUNDS)


def _wid():
    return lax.axis_index("s") * NC + lax.axis_index("c")


def _tc_zst_body(h_ref, wfc_ref, wa_ref, z_ref, st_ref):
    hb = h_ref[...]
    z = lax.dot_general(hb, wfc_ref[...], (((1,), (1,)), ((), ())),
                        preferred_element_type=_f32)
    z = jnp.where(z >= 0, z, 0.01 * z)
    z_ref[...] = z
    st = lax.dot_general(wa_ref[...], z, (((1,), (1,)), ((), ())),
                         preferred_element_type=_f32)
    st_ref[...] = jnp.concatenate(
        [st, jnp.zeros((6, st.shape[1]), _f32)], axis=0)


def _tc_zst(h_pad, wfc, wa2):
    BN = 512
    grid = NPAD // BN
    return pl.pallas_call(
        _tc_zst_body,
        grid=(grid,),
        in_specs=[
            pl.BlockSpec((BN, D), lambda i: (i, 0)),
            pl.BlockSpec((D, D), lambda i: (0, 0)),
            pl.BlockSpec((2, D), lambda i: (0, 0)),
        ],
        out_specs=[
            pl.BlockSpec((BN, D), lambda i: (i, 0)),
            pl.BlockSpec((8, BN), lambda i: (0, i)),
        ],
        out_shape=[
            jax.ShapeDtypeStruct((NPAD, D), _f32),
            jax.ShapeDtypeStruct((8, NPAD), _f32),
        ],
    )(h_pad, wfc, wa2)


def _tc_sum_body(p_ref, o_ref):
    o_ref[...] = p_ref[0] + p_ref[1]


def _tc_sum(part):
    BN = 512
    return pl.pallas_call(
        _tc_sum_body,
        grid=(NACC // BN,),
        in_specs=[pl.BlockSpec((2, BN, D), lambda i: (0, i, 0))],
        out_specs=pl.BlockSpec((BN, D), lambda i: (i, 0)),
        out_shape=jax.ShapeDtypeStruct((NACC, D), _f32),
    )(part)


@functools.partial(
    pl.kernel,
    out_type=(jax.ShapeDtypeStruct((E,), _f32),
              jax.ShapeDtypeStruct((NW * NPAD,), _f32),
              jax.ShapeDtypeStruct((NW * NPAD,), _f32)),
    mesh=plsc.VectorSubcoreMesh(**_MESH),
    compiler_params=pltpu.CompilerParams(needs_layout_passes=False),
    scratch_types=[
        pltpu.VMEM((NPAD,), _f32),
        pltpu.VMEM((NPAD,), _f32),
        pltpu.VMEM((NPAD,), _f32),
        pltpu.VMEM((NPAD,), _f32),
        pltpu.VMEM((EW,), _i32),
        pltpu.VMEM((EW,), _i32),
        pltpu.VMEM((EW,), _f32),
        pltpu.VMEM((EW,), _f32),
        pltpu.VMEM((16,), _f32),
        pltpu.VMEM((16,), _f32),
        pltpu.SemaphoreType.DMA,
    ],
)
def _k1(src_h, dst_h, ea_h, s_h, t_h, we_h, wm_h, exl_h, pmax_h, pden_h,
        s_v, t_v, m_v, d_v, src_v, dst_v, ea_v, ne_v, we_v, wm_v, sem):
    wid = _wid()
    base = wid * EW
    descs = [
        pltpu.async_copy(s_h, s_v, sem),
        pltpu.async_copy(t_h, t_v, sem),
        pltpu.async_copy(src_h.at[pl.ds(base, EW)], src_v, sem),
        pltpu.async_copy(dst_h.at[pl.ds(base, EW)], dst_v, sem),
        pltpu.async_copy(ea_h.at[pl.ds(base, EW)], ea_v, sem),
        pltpu.async_copy(we_h, we_v, sem),
        pltpu.async_copy(wm_h, wm_v, sem),
    ]

    def init_body(i, _):
        m_v[pl.ds(i * 16, 16)] = jnp.full((16,), -1e30, _f32)
        d_v[pl.ds(i * 16, 16)] = jnp.zeros((16,), _f32)
        return 0
    lax.fori_loop(0, NPAD // 16, init_body, 0)
    for de in descs:
        de.wait()
    we = we_v[...]
    wm = wm_v[...]

    def edge_body(i, _):
        sl = pl.ds(i * 16, 16)
        src16 = src_v[sl]
        dst16 = dst_v[sl]
        a = plsc.load_gather(s_v, [src16]) + plsc.load_gather(t_v, [dst16])
        x = a * ea_v[sl] * we
        ne = jnp.maximum(x, 0.01 * x) * wm
        ne_v[sl] = ne
        cur = plsc.load_gather(m_v, [dst16])
        need = ne > cur
        plsc.store_scatter(m_v, [dst16], ne, mask=need)

        def cond(p):
            return jnp.any(p)

        def body(p):
            cur2 = plsc.load_gather(m_v, [dst16])
            need2 = jnp.logical_and(p, ne > cur2)
            plsc.store_scatter(m_v, [dst16], ne, mask=need2)
            return need2
        lax.while_loop(cond, body, need)
        return 0
    lax.fori_loop(0, EW // 16, edge_body, 0)

    def exp_body(i, _):
        sl = pl.ds(i * 16, 16)
        dst16 = dst_v[sl]
        ex = jnp.exp(ne_v[sl] - plsc.load_gather(m_v, [dst16]))
        ne_v[sl] = ex
        plsc.addupdate_scatter(d_v, [dst16], ex)
        return 0
    lax.fori_loop(0, EW // 16, exp_body, 0)

    pltpu.sync_copy(ne_v, exl_h.at[pl.ds(base, EW)])
    pltpu.sync_copy(m_v, pmax_h.at[pl.ds(wid * NPAD, NPAD)])
    pltpu.sync_copy(d_v, pden_h.at[pl.ds(wid * NPAD, NPAD)])


@functools.partial(
    pl.kernel,
    out_type=jax.ShapeDtypeStruct((NW * NPAD,), _f32),
    mesh=plsc.VectorSubcoreMesh(**_MESH),
    compiler_params=pltpu.CompilerParams(needs_layout_passes=False),
    scratch_types=[
        pltpu.VMEM((NW * CH,), _f32),
        pltpu.VMEM((NW * CH,), _f32),
        pltpu.VMEM((NW * CH,), _f32),
        pltpu.SemaphoreType.DMA,
    ],
)
def _kred(pm_h, pd_h, scale_h, bm_v, bd_v, sc_v, sem):
    wid = _wid()
    descs = []
    for r in range(NW):
        descs.append(pltpu.async_copy(
            pm_h.at[pl.ds(r * NPAD + wid * CH, CH)],
            bm_v.at[pl.ds(r * CH, CH)], sem))
        descs.append(pltpu.async_copy(
            pd_h.at[pl.ds(r * NPAD + wid * CH, CH)],
            bd_v.at[pl.ds(r * CH, CH)], sem))
    for de in descs:
        de.wait()

    def body(j, _):
        sl = pl.ds(j * 16, 16)
        mx = bm_v[sl]
        for r in range(1, NW):
            mx = jnp.maximum(mx, bm_v[pl.ds(r * CH + j * 16, 16)])
        den = jnp.zeros((16,), _f32)
        for r in range(NW):
            rsl = pl.ds(r * CH + j * 16, 16)
            den = den + jnp.exp(bm_v[rsl] - mx) * bd_v[rsl]
        den = jnp.where(den > 0.0, den, 1.0)
        for r in range(NW):
            rsl = pl.ds(r * CH + j * 16, 16)
            sc_v[rsl] = jnp.exp(bm_v[rsl] - mx) / den
        return 0
    lax.fori_loop(0, CH // 16, body, 0)
    for r in range(NW):
        pltpu.sync_copy(sc_v.at[pl.ds(r * CH, CH)],
                        scale_h.at[pl.ds(r * NPAD + wid * CH, CH)])


@functools.partial(
    pl.kernel,
    out_type=jax.ShapeDtypeStruct((NW * EW2,), _f32),
    mesh=plsc.VectorSubcoreMesh(**_MESH),
    compiler_params=pltpu.CompilerParams(needs_layout_passes=False),
    scratch_types=[
        pltpu.VMEM((NPAD,), _f32),
        pltpu.VMEM((EW,), _i32),
        pltpu.VMEM((EW,), _f32),
        pltpu.VMEM((EW2,), _f32),
        pltpu.SemaphoreType.DMA,
    ],
)
def _k4(exl_h, dst_h, scale_h, al_h, scl_v, dst_v, ex_v, al_v, sem):
    wid = _wid()
    base = wid * EW
    descs = [
        pltpu.async_copy(scale_h.at[pl.ds(wid * NPAD, NPAD)], scl_v, sem),
        pltpu.async_copy(dst_h.at[pl.ds(base, EW)], dst_v, sem),
        pltpu.async_copy(exl_h.at[pl.ds(base, EW)], ex_v, sem),
    ]
    for i in range((EW2 - EW) // 16):
        al_v[pl.ds(EW + i * 16, 16)] = jnp.zeros((16,), _f32)
    for de in descs:
        de.wait()

    def body(i, _):
        sl = pl.ds(i * 16, 16)
        dst16 = dst_v[sl]
        al_v[sl] = ex_v[sl] * plsc.load_gather(scl_v, [dst16])
        return 0
    lax.fori_loop(0, EW // 16, body, 0)
    pltpu.sync_copy(al_v, al_h.at[pl.ds(wid * EW2, EW2)])


@functools.partial(
    pl.kernel,
    out_type=jax.ShapeDtypeStruct((NC, NACC, D), _f32),
    mesh=plsc.VectorSubcoreMesh(**_MESH),
    compiler_params=pltpu.CompilerParams(needs_layout_passes=False),
    scratch_types=[
        pltpu.VMEM((EW2,), _f32),
        pltpu.VMEM((2, B), _i32),
        pltpu.VMEM((2, B), _i32),
        pltpu.VMEM((B, D), _f32),
        pltpu.VMEM((B, D), _f32),
        pltpu.VMEM_SHARED((NACC, D), _f32),
        pltpu.SemaphoreType.DMA,
        pltpu.SemaphoreType.DMA,
        pltpu.SemaphoreType.DMA,
        pltpu.SemaphoreType.DMA,
        pltpu.SemaphoreType.DMA,
        pltpu.SemaphoreType.DMA,
        pltpu.SemaphoreType.DMA,
        pltpu.SemaphoreType.DMA,
    ],
)
def _k5(al_h, src_h, dst_h, z_h, part_h,
        al_v, sidx_v, didx_v, rows0_v, rows1_v, acc_sh,
        gs0, gs1, ss0, ss1, is0, is1, ds0, ds1):
    cid = lax.axis_index("c")
    sid = lax.axis_index("s")
    wid = sid * NC + cid
    rows = (rows0_v, rows1_v)
    gsem = (gs0, gs1)
    ssem = (ss0, ss1)
    isem = (is0, is1)
    dsem = (ds0, ds1)
    base2 = wid * EW2

    def _erow(h, j):
        return h.at[pl.ds(base2 + j * B, B)]

    pltpu.sync_copy(al_h.at[pl.ds(base2, EW2)], al_v)
    for b in range(2):
        pltpu.sync_copy(_erow(src_h, b), sidx_v.at[b])
        pltpu.async_copy(_erow(dst_h, b), didx_v.at[b], dsem[b])

    def zinit(i, _):
        for r in range(ZR):
            rows0_v[r, pl.ds(i * 16, 16)] = jnp.zeros((16,), _f32)
        return 0
    lax.fori_loop(0, D // 16, zinit, 0)
    row0 = sid * RPW

    def zcopy(q, _):
        pltpu.sync_copy(rows0_v.at[pl.ds(0, ZR)],
                        acc_sh.at[pl.ds(row0 + q * ZR, ZR)])
        return 0
    lax.fori_loop(0, RPW // ZR, zcopy, 0)
    plsc.subcore_barrier()

    for b in range(2):
        pltpu.async_copy(z_h.at[sidx_v.at[b]], rows[b], gsem[b])

    def _scale(b, jb):
        for k in range(B // 16):
            al = al_v[pl.ds(jb * B + k * 16, 16)]
            for i in range(16):
                bc = _lane_bcast(al, i)
                r = k * 16 + i
                for c in range(D // 16):
                    cs = pl.ds(c * 16, 16)
                    rows[b][r, cs] = rows[b][r, cs] * bc

    def _pair(j, with_next):
        sdesc = [None, None]
        idescs = [None, None]
        for b in range(2):
            jb = j + b
            pltpu.make_async_copy(z_h.at[sidx_v.at[b]], rows[b],
                                  gsem[b]).wait()
            if with_next:
                idescs[b] = pltpu.async_copy(_erow(src_h, jb + 2),
                                             sidx_v.at[b], isem[b])
            _scale(b, jb)
            pltpu.make_async_copy(_erow(dst_h, 0), didx_v.at[b],
                                  dsem[b]).wait()
            sdesc[b] = pltpu.async_copy(rows[b], acc_sh.at[didx_v.at[b]],
                                        ssem[b], add=True)
        for b in range(2):
            sdesc[b].wait()
            if with_next:
                pltpu.async_copy(_erow(dst_h, j + b + 2), didx_v.at[b],
                                 dsem[b])
                idescs[b].wait()
                pltpu.async_copy(z_h.at[sidx_v.at[b]], rows[b], gsem[b])

    def pair_body(i, _):
        _pair(2 * i, True)
        return 0
    lax.fori_loop(0, NCHUNK // 2 - 1, pair_body, 0)
    _pair(NCHUNK - 2, False)

    plsc.subcore_barrier()

    def dump(q, _):
        sl = pl.ds(row0 + q * ZR, ZR)
        pltpu.sync_copy(acc_sh.at[sl], part_h.at[cid].at[sl])
        return 0
    lax.fori_loop(0, RPW // ZR, dump, 0)


def kernel(h, edge_index, edge_attr, W_fc, W_attn, W_edge, W_m):
    h_pad = jnp.pad(h, ((0, NPAD - N), (0, 0)))
    z, st8 = _tc_zst(h_pad, W_fc, W_attn.reshape(2, D))
    s = st8[0]
    t = st8[1]
    src = edge_index[0]
    dst = edge_index[1]
    ea = edge_attr[:, 0]
    we = jnp.broadcast_to(W_edge.reshape(()), (16,)).astype(_f32)
    wm = jnp.broadcast_to(W_m.reshape(()), (16,)).astype(_f32)

    exl, pmax, pden = _k1(src, dst, ea, s, t, we, wm)
    scale = _kred(pmax, pden)
    al2 = _k4(exl, dst, scale)

    npad_e = EW2 - EW
    pad_idx = (jnp.arange(npad_e, dtype=_i32) * 97) % N
    pad_blk = jnp.broadcast_to(pad_idx, (NW, npad_e))

    def _pad_edges(x):
        return jnp.concatenate([x.reshape(NW, EW), pad_blk],
                               axis=1).reshape(-1)

    part = _k5(al2, _pad_edges(src), _pad_edges(dst), z)
    out = _tc_sum(part)
    return out[:N]
